# baseline (speedup 1.0000x reference)
import functools

import jax
import jax.numpy as jnp
from jax import lax
from jax.experimental import pallas as pl
from jax.experimental.pallas import tpu as pltpu
from jax.experimental.pallas import tpu_sc as plsc

_NC = 2
_NS = 16
_L = 16
_NW = _NC * _NS

_B, _C = 2, 16
_N_IN = 12 * 256 * 256
_N_OUT = _N_IN // 4
_TOT_IN = _B * _C * _N_IN
_TOT_OUT = _TOT_IN // 4

_IN_W = _TOT_IN // _NW
_OUT_W = _IN_W // 4
_CH_IN = 49152
_CH_OUT = _CH_IN // 4
_NCHUNK = _IN_W // _CH_IN
_G = _CH_OUT // _L
_NBUF = 2


def _body(x_hbm, out_hbm, in_v0, in_v1, out_v0, out_v1,
          isem0, isem1, osem0, osem1):
    wid = lax.axis_index("s") * _NC + lax.axis_index("c")
    lanes4 = jnp.arange(_L, dtype=jnp.int32) * 4

    ins = [in_v0, in_v1]
    isems = [isem0, isem1]
    outs = [out_v0, out_v1]
    osems = [osem0, osem1]
    in_cp = [None] * _NBUF
    out_cp = [None, None]

    for k in range(_NBUF - 1):
        in_cp[k] = pltpu.async_copy(
            x_hbm.at[wid, pl.ds(k * _CH_IN, _CH_IN)], ins[k], isems[k])

    for k in range(_NCHUNK):
        b = k % _NBUF
        ob = k & 1
        nk = k + _NBUF - 1
        if nk < _NCHUNK:
            nb = nk % _NBUF
            in_cp[nb] = pltpu.async_copy(
                x_hbm.at[wid, pl.ds(nk * _CH_IN, _CH_IN)],
                ins[nb], isems[nb])
        in_cp[b].wait()
        if out_cp[ob] is not None:
            out_cp[ob].wait()

        in_v = ins[b]
        out_v = outs[ob]

        @plsc.parallel_loop(0, _G, 1, unroll=8)
        def _group(g):
            i0 = lanes4 + g * 64
            v = plsc.load_gather(in_v, [i0])
            v = jnp.maximum(v, plsc.load_gather(in_v, [i0 + 1]))
            v = jnp.maximum(v, plsc.load_gather(in_v, [i0 + 2]))
            v = jnp.maximum(v, plsc.load_gather(in_v, [i0 + 3]))
            out_v[pl.ds(g * _L, _L)] = v

        out_cp[ob] = pltpu.async_copy(
            out_v, out_hbm.at[wid, pl.ds(k * _CH_OUT, _CH_OUT)],
            osems[ob])

    for cp in out_cp:
        if cp is not None:
            cp.wait()


_maxpool4 = functools.partial(
    pl.kernel,
    out_type=jax.ShapeDtypeStruct((_NW, _OUT_W), jnp.float32),
    mesh=plsc.VectorSubcoreMesh(core_axis_name="c", subcore_axis_name="s"),
    scratch_types=[
        pltpu.VMEM((_CH_IN,), jnp.float32),
        pltpu.VMEM((_CH_IN,), jnp.float32),
        pltpu.VMEM((_CH_OUT,), jnp.float32),
        pltpu.VMEM((_CH_OUT,), jnp.float32),
        pltpu.SemaphoreType.DMA,
        pltpu.SemaphoreType.DMA,
        pltpu.SemaphoreType.DMA,
        pltpu.SemaphoreType.DMA,
    ],
    compiler_params=pltpu.CompilerParams(needs_layout_passes=False),
)(_body)


@jax.jit
def kernel(x, children):
    del children
    y2 = _maxpool4(x.reshape(_NW, _N_IN))
    return y2.reshape(_B, _C, _N_OUT)


# speedup vs baseline: 1.0098x; 1.0098x over previous
import functools

import jax
import jax.numpy as jnp
from jax import lax
from jax.experimental import pallas as pl
from jax.experimental.pallas import tpu as pltpu
from jax.experimental.pallas import tpu_sc as plsc

_NC = 2
_NS = 16
_L = 16
_NW = _NC * _NS

_B, _C = 2, 16
_N_IN = 12 * 256 * 256
_N_OUT = _N_IN // 4
_TOT_IN = _B * _C * _N_IN
_TOT_OUT = _TOT_IN // 4

_IN_W = _TOT_IN // _NW
_OUT_W = _IN_W // 4
_CH_IN = 32768
_CH_OUT = _CH_IN // 4
_NCHUNK = _IN_W // _CH_IN
_G = _CH_OUT // _L
_NBUF = 3


def _body(x_hbm, out_hbm, in_v0, in_v1, in_v2, out_v0, out_v1,
          isem0, isem1, isem2, osem0, osem1):
    wid = lax.axis_index("s") * _NC + lax.axis_index("c")
    lanes4 = jnp.arange(_L, dtype=jnp.int32) * 4

    ins = [in_v0, in_v1, in_v2]
    isems = [isem0, isem1, isem2]
    outs = [out_v0, out_v1]
    osems = [osem0, osem1]
    in_cp = [None] * _NBUF
    out_cp = [None, None]

    for k in range(_NBUF - 1):
        in_cp[k] = pltpu.async_copy(
            x_hbm.at[wid, pl.ds(k * _CH_IN, _CH_IN)], ins[k], isems[k])

    for k in range(_NCHUNK):
        b = k % _NBUF
        ob = k & 1
        nk = k + _NBUF - 1
        if nk < _NCHUNK:
            nb = nk % _NBUF
            in_cp[nb] = pltpu.async_copy(
                x_hbm.at[wid, pl.ds(nk * _CH_IN, _CH_IN)],
                ins[nb], isems[nb])
        in_cp[b].wait()
        if out_cp[ob] is not None:
            out_cp[ob].wait()

        in_v = ins[b]
        out_v = outs[ob]

        @plsc.parallel_loop(0, _G, 1, unroll=8)
        def _group(g):
            i0 = lanes4 + g * 64
            v = plsc.load_gather(in_v, [i0])
            v = jnp.maximum(v, plsc.load_gather(in_v, [i0 + 1]))
            v = jnp.maximum(v, plsc.load_gather(in_v, [i0 + 2]))
            v = jnp.maximum(v, plsc.load_gather(in_v, [i0 + 3]))
            out_v[pl.ds(g * _L, _L)] = v

        out_cp[ob] = pltpu.async_copy(
            out_v, out_hbm.at[wid, pl.ds(k * _CH_OUT, _CH_OUT)],
            osems[ob])

    for cp in out_cp:
        if cp is not None:
            cp.wait()


_maxpool4 = functools.partial(
    pl.kernel,
    out_type=jax.ShapeDtypeStruct((_NW, _OUT_W), jnp.float32),
    mesh=plsc.VectorSubcoreMesh(core_axis_name="c", subcore_axis_name="s"),
    scratch_types=[
        pltpu.VMEM((_CH_IN,), jnp.float32),
        pltpu.VMEM((_CH_IN,), jnp.float32),
        pltpu.VMEM((_CH_IN,), jnp.float32),
        pltpu.VMEM((_CH_OUT,), jnp.float32),
        pltpu.VMEM((_CH_OUT,), jnp.float32),
        pltpu.SemaphoreType.DMA,
        pltpu.SemaphoreType.DMA,
        pltpu.SemaphoreType.DMA,
        pltpu.SemaphoreType.DMA,
        pltpu.SemaphoreType.DMA,
    ],
    compiler_params=pltpu.CompilerParams(needs_layout_passes=False),
)(_body)


@jax.jit
def kernel(x, children):
    del children
    y2 = _maxpool4(x.reshape(_NW, _N_IN))
    return y2.reshape(_B, _C, _N_OUT)
